# baseline (speedup 1.0000x reference)
import functools

import jax
import jax.numpy as jnp
from jax import lax
from jax.experimental import pallas as pl
from jax.experimental.pallas import tpu as pltpu
from jax.experimental.pallas import tpu_sc as plsc

_NC = 2
_NS = 16
_NW = _NC * _NS
_L = 200
_CR = 16
_NBUF = 4


def _proj_body(t_ref, w_ref, o_ref):
    o_ref[...] = jax.lax.dot_general(
        t_ref[...], w_ref[...], (((1,), (0,)), ((), ())),
        preferred_element_type=jnp.float32,
        precision=jax.lax.Precision.HIGHEST)


def _project_table(table, fc_w):
    V, D = table.shape
    nl = fc_w.shape[0]
    wt = jnp.zeros((D, 16), jnp.float32).at[:, :nl].set(fc_w.T)
    blk = 4000
    assert V % blk == 0
    return pl.pallas_call(
        _proj_body,
        grid=(V // blk,),
        in_specs=[
            pl.BlockSpec((blk, D), lambda i: (i, 0)),
            pl.BlockSpec((D, 16), lambda i: (0, 0)),
        ],
        out_specs=pl.BlockSpec((blk, 16), lambda i: (i, 0)),
        out_shape=jax.ShapeDtypeStruct((V, 16), jnp.float32),
    )(table, wt)


@functools.lru_cache(maxsize=2)
def _make_sc_bag(B):
    rows_per_w = B // _NW
    n_chunks = rows_per_w // _CR
    mesh = plsc.VectorSubcoreMesh(core_axis_name="c", subcore_axis_name="s")

    @functools.partial(
        pl.kernel,
        out_type=jax.ShapeDtypeStruct((B, 16), jnp.float32),
        mesh=mesh,
        compiler_params=pltpu.CompilerParams(use_tc_tiling_on_sc=False),
        scratch_types=[
            pltpu.VMEM((2, _CR, _L), jnp.int32),
            pltpu.VMEM((_NBUF, _L, 16), jnp.float32),
            pltpu.VMEM((_CR, 16), jnp.float32),
            pltpu.VMEM((8, 16), jnp.float32),
            pltpu.VMEM((16,), jnp.float32),
            pltpu.SemaphoreType.DMA,
            pltpu.SemaphoreType.DMA,
            pltpu.SemaphoreType.DMA,
            pltpu.SemaphoreType.DMA,
            pltpu.SemaphoreType.DMA,
        ],
    )
    def sc_bag(text_hbm, proj_hbm, bias_hbm, out_hbm,
               idx_v, rows_v, out_v, p0_v, bias_v,
               isem, rsem0, rsem1, rsem2, rsem3):
        wid = lax.axis_index("s") * _NC + lax.axis_index("c")
        base = wid * rows_per_w
        rsems = (rsem0, rsem1, rsem2, rsem3)

        pltpu.sync_copy(proj_hbm.at[pl.ds(0, 8)], p0_v)
        pltpu.sync_copy(bias_hbm, bias_v)
        p0 = p0_v[0, :]
        bias = bias_v[:]
        lane = lax.iota(jnp.int32, 16)

        def gather_row(par, r, b):
            c1 = pltpu.async_copy(
                proj_hbm.at[idx_v.at[par, r, pl.ds(0, 128)]],
                rows_v.at[b, pl.ds(0, 128)], rsems[b])
            c2 = pltpu.async_copy(
                proj_hbm.at[idx_v.at[par, r, pl.ds(128, _L - 128)]],
                rows_v.at[b, pl.ds(128, _L - 128)], rsems[b])
            return (c1, c2)

        def accum_row(b):
            zero = jnp.zeros((16,), jnp.float32)

            def body(i, accs):
                l = i * 8
                return tuple(accs[j] + rows_v[b, l + j, :] for j in range(8))

            accs = lax.fori_loop(0, _L // 8, body, (zero,) * 8)
            a0 = (accs[0] + accs[1]) + (accs[2] + accs[3])
            a1 = (accs[4] + accs[5]) + (accs[6] + accs[7])
            return a0 + a1

        def count_nonzero(par, r):
            ones = jnp.ones((16,), jnp.float32)
            zeros = jnp.zeros((16,), jnp.float32)
            cv = zeros
            for c in range(_L // 16):
                v = idx_v[par, r, pl.ds(c * 16, 16)]
                cv = cv + jnp.where(v != 0, ones, zeros)
            vt = idx_v[par, r, pl.ds(_L - 16, 16)]
            cv = cv + jnp.where((vt != 0) & (lane >= 8), ones, zeros)
            parts = [cv[i] for i in range(16)]
            while len(parts) > 1:
                parts = [parts[i] + parts[i + 1]
                         for i in range(0, len(parts), 2)]
            return parts[0]

        def do_chunk(ci, par):
            row0 = base + ci * _CR
            nci = ci + 1

            @pl.when(nci < n_chunks)
            def _():
                pltpu.async_copy(
                    text_hbm.at[pl.ds(base + nci * _CR, _CR)],
                    idx_v.at[(par + 1) % 2], isem)

            pend = gather_row(par, 0, 0)
            for r in range(_CR):
                if r + 1 < _CR:
                    nxt = gather_row(par, r + 1, (r + 1) % _NBUF)
                for cp in pend:
                    cp.wait()
                acc = accum_row(r % _NBUF)
                cntf = count_nonzero(par, r)
                n0f = jnp.float32(_L) - cntf
                denom = jnp.maximum(jnp.zeros((16,), jnp.float32) + cntf, 1.0)
                res = (acc - n0f * p0) / denom + bias
                out_v[r, :] = res
                if r + 1 < _CR:
                    pend = nxt
            pltpu.sync_copy(out_v, out_hbm.at[pl.ds(row0, _CR)])

            @pl.when(nci < n_chunks)
            def _():
                pltpu.make_async_copy(
                    text_hbm.at[pl.ds(base, _CR)],
                    idx_v.at[(par + 1) % 2], isem).wait()

        pltpu.sync_copy(text_hbm.at[pl.ds(base, _CR)], idx_v.at[0])

        def two_chunks(k, _):
            do_chunk(k * 2, 0)
            do_chunk(k * 2 + 1, 1)
            return 0

        lax.fori_loop(0, n_chunks // 2, two_chunks, 0)

    return sc_bag


def kernel(text, limits, emb_table, fc_w, fc_b):
    del limits
    B, L = text.shape
    assert L == _L
    proj = _project_table(emb_table, fc_w)
    bias = jnp.zeros((16,), jnp.float32).at[: fc_b.shape[0]].set(fc_b)
    out16 = _make_sc_bag(B)(text, proj, bias)
    return out16[:, : fc_b.shape[0]]


# speedup vs baseline: 1.3712x; 1.3712x over previous
import functools

import jax
import jax.numpy as jnp
from jax import lax
from jax.experimental import pallas as pl
from jax.experimental.pallas import tpu as pltpu
from jax.experimental.pallas import tpu_sc as plsc

_NC = 2
_NS = 16
_NW = _NC * _NS
_L = 200
_CR = 16
_NBUF = 4


def _proj_body(t_ref, w_ref, o_ref):
    o_ref[...] = jax.lax.dot_general(
        t_ref[...], w_ref[...], (((1,), (0,)), ((), ())),
        preferred_element_type=jnp.float32,
        precision=jax.lax.Precision.HIGHEST)


def _project_table(table, fc_w):
    V, D = table.shape
    nl = fc_w.shape[0]
    wt = jnp.zeros((D, 16), jnp.float32).at[:, :nl].set(fc_w.T)
    wblk = jnp.kron(jnp.eye(4, dtype=jnp.float32), wt)
    t4 = table.reshape(V // 4, 4 * D)
    blk = 5000
    assert (V // 4) % blk == 0
    return pl.pallas_call(
        _proj_body,
        grid=(V // 4 // blk,),
        in_specs=[
            pl.BlockSpec((blk, 4 * D), lambda i: (i, 0)),
            pl.BlockSpec((4 * D, 64), lambda i: (0, 0)),
        ],
        out_specs=pl.BlockSpec((blk, 64), lambda i: (i, 0)),
        out_shape=jax.ShapeDtypeStruct((V // 4, 64), jnp.float32),
    )(t4, wblk).reshape(V, 16)


@functools.lru_cache(maxsize=2)
def _make_sc_bag(B):
    rows_per_w = B // _NW
    n_chunks = rows_per_w // _CR
    mesh = plsc.VectorSubcoreMesh(core_axis_name="c", subcore_axis_name="s")

    @functools.partial(
        pl.kernel,
        out_type=jax.ShapeDtypeStruct((B, 16), jnp.float32),
        mesh=mesh,
        compiler_params=pltpu.CompilerParams(use_tc_tiling_on_sc=False),
        scratch_types=[
            pltpu.VMEM((2, _CR, _L), jnp.int32),
            pltpu.VMEM((_NBUF, _L, 16), jnp.float32),
            pltpu.VMEM((_CR, 16), jnp.float32),
            pltpu.VMEM((8, 16), jnp.float32),
            pltpu.VMEM((16,), jnp.float32),
            pltpu.SemaphoreType.DMA,
            pltpu.SemaphoreType.DMA,
            pltpu.SemaphoreType.DMA,
            pltpu.SemaphoreType.DMA,
            pltpu.SemaphoreType.DMA,
        ],
    )
    def sc_bag(text_hbm, proj_hbm, bias_hbm, out_hbm,
               idx_v, rows_v, out_v, p0_v, bias_v,
               isem, rsem0, rsem1, rsem2, rsem3):
        wid = lax.axis_index("s") * _NC + lax.axis_index("c")
        base = wid * rows_per_w
        rsems = (rsem0, rsem1, rsem2, rsem3)

        pltpu.sync_copy(proj_hbm.at[pl.ds(0, 8)], p0_v)
        pltpu.sync_copy(bias_hbm, bias_v)
        p0 = p0_v[0, :]
        bias = bias_v[:]
        lane = lax.iota(jnp.int32, 16)

        def gather_row(par, r, b):
            c1 = pltpu.async_copy(
                proj_hbm.at[idx_v.at[par, r, pl.ds(0, 128)]],
                rows_v.at[b, pl.ds(0, 128)], rsems[b])
            c2 = pltpu.async_copy(
                proj_hbm.at[idx_v.at[par, r, pl.ds(128, _L - 128)]],
                rows_v.at[b, pl.ds(128, _L - 128)], rsems[b])
            return (c1, c2)

        def accum_row(b):
            zero = jnp.zeros((16,), jnp.float32)

            def body(i, accs):
                l = i * 8
                return tuple(accs[j] + rows_v[b, l + j, :] for j in range(8))

            accs = lax.fori_loop(0, _L // 8, body, (zero,) * 8)
            a0 = (accs[0] + accs[1]) + (accs[2] + accs[3])
            a1 = (accs[4] + accs[5]) + (accs[6] + accs[7])
            return a0 + a1

        def count_nonzero(par, r):
            ones = jnp.ones((16,), jnp.float32)
            zeros = jnp.zeros((16,), jnp.float32)
            cv = zeros
            for c in range(_L // 16):
                v = idx_v[par, r, pl.ds(c * 16, 16)]
                cv = cv + jnp.where(v != 0, ones, zeros)
            vt = idx_v[par, r, pl.ds(_L - 16, 16)]
            cv = cv + jnp.where((vt != 0) & (lane >= 8), ones, zeros)
            parts = [cv[i] for i in range(16)]
            while len(parts) > 1:
                parts = [parts[i] + parts[i + 1]
                         for i in range(0, len(parts), 2)]
            return parts[0]

        def do_chunk(ci, par):
            row0 = base + ci * _CR
            nci = ci + 1

            @pl.when(nci < n_chunks)
            def _():
                pltpu.async_copy(
                    text_hbm.at[pl.ds(base + nci * _CR, _CR)],
                    idx_v.at[(par + 1) % 2], isem)

            pend = [gather_row(par, 0, 0), gather_row(par, 1, 1)]
            for r in range(_CR):
                if r + 2 < _CR:
                    pend.append(gather_row(par, r + 2, (r + 2) % _NBUF))
                for cp in pend[r]:
                    cp.wait()
                acc = accum_row(r % _NBUF)
                cntf = count_nonzero(par, r)
                n0f = jnp.float32(_L) - cntf
                denom = jnp.maximum(jnp.zeros((16,), jnp.float32) + cntf, 1.0)
                res = (acc - n0f * p0) / denom + bias
                out_v[r, :] = res
            pltpu.sync_copy(out_v, out_hbm.at[pl.ds(row0, _CR)])

            @pl.when(nci < n_chunks)
            def _():
                pltpu.make_async_copy(
                    text_hbm.at[pl.ds(base, _CR)],
                    idx_v.at[(par + 1) % 2], isem).wait()

        pltpu.sync_copy(text_hbm.at[pl.ds(base, _CR)], idx_v.at[0])

        def two_chunks(k, _):
            do_chunk(k * 2, 0)
            do_chunk(k * 2 + 1, 1)
            return 0

        lax.fori_loop(0, n_chunks // 2, two_chunks, 0)

    return sc_bag


def kernel(text, limits, emb_table, fc_w, fc_b):
    del limits
    B, L = text.shape
    assert L == _L
    proj = _project_table(emb_table, fc_w)
    bias = jnp.zeros((16,), jnp.float32).at[: fc_b.shape[0]].set(fc_b)
    out16 = _make_sc_bag(B)(text, proj, bias)
    return out16[:, : fc_b.shape[0]]


# speedup vs baseline: 1.4176x; 1.0338x over previous
import functools

import jax
import jax.numpy as jnp
from jax import lax
from jax.experimental import pallas as pl
from jax.experimental.pallas import tpu as pltpu
from jax.experimental.pallas import tpu_sc as plsc

_NC = 2
_NS = 16
_NW = _NC * _NS
_L = 200
_CR = 16
_NBUF = 4


def _proj_body(t_ref, w_ref, o_ref):
    o_ref[...] = jax.lax.dot_general(
        t_ref[...], w_ref[...], (((1,), (0,)), ((), ())),
        preferred_element_type=jnp.float32,
        precision=jax.lax.Precision.HIGHEST)


def _project_table(table, fc_w):
    V, D = table.shape
    nl = fc_w.shape[0]
    wt = jnp.zeros((D, 16), jnp.float32).at[:, :nl].set(fc_w.T)
    wblk = jnp.kron(jnp.eye(4, dtype=jnp.float32), wt)
    t4 = table.reshape(V // 4, 4 * D)
    blk = 5000
    assert (V // 4) % blk == 0
    return pl.pallas_call(
        _proj_body,
        grid=(V // 4 // blk,),
        in_specs=[
            pl.BlockSpec((blk, 4 * D), lambda i: (i, 0)),
            pl.BlockSpec((4 * D, 64), lambda i: (0, 0)),
        ],
        out_specs=pl.BlockSpec((blk, 64), lambda i: (i, 0)),
        out_shape=jax.ShapeDtypeStruct((V // 4, 64), jnp.float32),
    )(t4, wblk).reshape(V, 16)


@functools.lru_cache(maxsize=2)
def _make_sc_bag(B):
    rows_per_w = B // _NW
    n_chunks = rows_per_w // _CR
    mesh = plsc.VectorSubcoreMesh(core_axis_name="c", subcore_axis_name="s")

    @functools.partial(
        pl.kernel,
        out_type=jax.ShapeDtypeStruct((B, 16), jnp.float32),
        mesh=mesh,
        compiler_params=pltpu.CompilerParams(use_tc_tiling_on_sc=False),
        scratch_types=[
            pltpu.VMEM((2, _CR, _L), jnp.int32),
            pltpu.VMEM((_NBUF, _L, 16), jnp.float32),
            pltpu.VMEM((_CR, 16), jnp.float32),
            pltpu.VMEM((8, 16), jnp.float32),
            pltpu.VMEM((16,), jnp.float32),
            pltpu.SemaphoreType.DMA,
            pltpu.SemaphoreType.DMA,
            pltpu.SemaphoreType.DMA,
            pltpu.SemaphoreType.DMA,
            pltpu.SemaphoreType.DMA,
        ],
    )
    def sc_bag(text_hbm, proj_hbm, bias_hbm, out_hbm,
               idx_v, rows_v, out_v, p0_v, bias_v,
               isem, rsem0, rsem1, rsem2, rsem3):
        wid = lax.axis_index("s") * _NC + lax.axis_index("c")
        base = wid * rows_per_w
        rsems = (rsem0, rsem1, rsem2, rsem3)

        pltpu.sync_copy(proj_hbm.at[pl.ds(0, 8)], p0_v)
        pltpu.sync_copy(bias_hbm, bias_v)
        p0 = p0_v[0, :]
        bias = bias_v[:]
        lane = lax.iota(jnp.int32, 16)

        def gather_row(par, r, b):
            c1 = pltpu.async_copy(
                proj_hbm.at[idx_v.at[par, r, pl.ds(0, 128)]],
                rows_v.at[b, pl.ds(0, 128)], rsems[b])
            c2 = pltpu.async_copy(
                proj_hbm.at[idx_v.at[par, r, pl.ds(128, _L - 128)]],
                rows_v.at[b, pl.ds(128, _L - 128)], rsems[b])
            return (c1, c2)

        def accum_row(b):
            zero = jnp.zeros((16,), jnp.float32)

            def body(i, accs):
                l = i * 8
                return tuple(accs[j] + rows_v[b, l + j, :] for j in range(8))

            accs = lax.fori_loop(0, _L // 8, body, (zero,) * 8)
            a0 = (accs[0] + accs[1]) + (accs[2] + accs[3])
            a1 = (accs[4] + accs[5]) + (accs[6] + accs[7])
            return a0 + a1

        def count_nonzero(par, r):
            ones = jnp.ones((16,), jnp.float32)
            zeros = jnp.zeros((16,), jnp.float32)
            cv = zeros
            for c in range(_L // 16):
                v = idx_v[par, r, pl.ds(c * 16, 16)]
                cv = cv + jnp.where(v != 0, ones, zeros)
            vt = idx_v[par, r, pl.ds(_L - 16, 16)]
            cv = cv + jnp.where((vt != 0) & (lane >= 8), ones, zeros)
            parts = [cv[i] for i in range(16)]
            while len(parts) > 1:
                parts = [parts[i] + parts[i + 1]
                         for i in range(0, len(parts), 2)]
            return parts[0]

        def do_chunk(ci, par):
            row0 = base + ci * _CR
            nci = ci + 1

            @pl.when(nci < n_chunks)
            def _():
                pltpu.async_copy(
                    text_hbm.at[pl.ds(base + nci * _CR, _CR)],
                    idx_v.at[(par + 1) % 2], isem)

            pend = [gather_row(par, 0, 0), gather_row(par, 1, 1),
                    gather_row(par, 2, 2)]
            for r in range(_CR):
                if r + 3 < _CR:
                    pend.append(gather_row(par, r + 3, (r + 3) % _NBUF))
                for cp in pend[r]:
                    cp.wait()
                acc = accum_row(r % _NBUF)
                cntf = count_nonzero(par, r)
                n0f = jnp.float32(_L) - cntf
                denom = jnp.maximum(jnp.zeros((16,), jnp.float32) + cntf, 1.0)
                res = (acc - n0f * p0) / denom + bias
                out_v[r, :] = res
            pltpu.sync_copy(out_v, out_hbm.at[pl.ds(row0, _CR)])

            @pl.when(nci < n_chunks)
            def _():
                pltpu.make_async_copy(
                    text_hbm.at[pl.ds(base, _CR)],
                    idx_v.at[(par + 1) % 2], isem).wait()

        pltpu.sync_copy(text_hbm.at[pl.ds(base, _CR)], idx_v.at[0])

        def two_chunks(k, _):
            do_chunk(k * 2, 0)
            do_chunk(k * 2 + 1, 1)
            return 0

        lax.fori_loop(0, n_chunks // 2, two_chunks, 0)

    return sc_bag


def kernel(text, limits, emb_table, fc_w, fc_b):
    del limits
    B, L = text.shape
    assert L == _L
    proj = _project_table(emb_table, fc_w)
    bias = jnp.zeros((16,), jnp.float32).at[: fc_b.shape[0]].set(fc_b)
    out16 = _make_sc_bag(B)(text, proj, bias)
    return out16[:, : fc_b.shape[0]]
